# baseline (speedup 1.0000x reference)
import jax
import jax.numpy as jnp
from jax import lax
from jax.experimental import pallas as pl
from jax.experimental.pallas import tpu as pltpu
from jax.experimental.pallas import tpu_sc as plsc

N = 10000
NP = 10112
E = 320000
EP = 327680
D = 128
NCORES = 2
NSUB = 16
CHUNK = 128
NCHUNKS_TOTAL = EP // CHUNK
NCHUNK0 = 160
NCHUNK1 = NCHUNKS_TOTAL // NSUB - NCHUNK0
NCHUNK = EP // (NCORES * NSUB * CHUNK)
ROWS_PER_TILE = NP // NSUB
NBUF = 2
DEGW = 128

_F32 = jnp.float32


def _sc_mesh():
  return plsc.VectorSubcoreMesh(core_axis_name="c", subcore_axis_name="s")


def _sc_aggregate(hw, idx2d, zeros_big):
  out_types = jax.ShapeDtypeStruct((NCORES, NP, D), _F32)

  scratch = [
      pltpu.VMEM((NBUF, 2, CHUNK), jnp.int32),
      pltpu.VMEM((NBUF, CHUNK, D), _F32),
      pltpu.VMEM_SHARED((NP, D), _F32),
      pltpu.SemaphoreType.DMA((NBUF,)),
  ]

  def body(hw_hbm, idx_hbm, zb_hbm, acc_out, idx_v, rows, acc_sh, sems):
    cid = lax.axis_index("c")
    sid = lax.axis_index("s")
    rb = sid * ROWS_PER_TILE

    pltpu.sync_copy(zb_hbm.at[pl.ds(rb, ROWS_PER_TILE)],
                    acc_sh.at[pl.ds(rb, ROWS_PER_TILE)])

    plsc.subcore_barrier()

    def run_ring(ebase, nchunk):
      if nchunk == 0:
        return
      for b in range(NBUF):
        pltpu.sync_copy(idx_hbm.at[ebase + b], idx_v.at[b])
        pltpu.async_copy(hw_hbm.at[idx_v.at[b, 0]], rows.at[b], sems.at[b])

      @pl.loop(0, nchunk, step=NBUF)
      def _(g):
        for b in range(NBUF):
          ci = g + b
          pltpu.make_async_copy(
              hw_hbm.at[idx_v.at[b, 0]], rows.at[b], sems.at[b]).wait()
          pltpu.sync_copy(rows.at[b], acc_sh.at[idx_v.at[b, 1]], add=True)

          nci = ci + NBUF

          @pl.when(nci < nchunk)
          def _():
            pltpu.sync_copy(idx_hbm.at[ebase + nci], idx_v.at[b])
            pltpu.async_copy(hw_hbm.at[idx_v.at[b, 0]], rows.at[b],
                             sems.at[b])

    @pl.when(cid == 0)
    def _():
      run_ring(sid * NCHUNK0, NCHUNK0)

    @pl.when(cid == 1)
    def _():
      run_ring(NSUB * NCHUNK0 + sid * NCHUNK1, NCHUNK1)

    plsc.subcore_barrier()

    pltpu.sync_copy(acc_sh.at[pl.ds(rb, ROWS_PER_TILE)],
                    acc_out.at[cid, pl.ds(rb, ROWS_PER_TILE)])

  k = pl.kernel(body, out_type=out_types, mesh=_sc_mesh(),
                scratch_types=scratch)
  return k(hw, idx2d, zeros_big)


def _sc_degree(dst2d, zeros_deg, ones_deg):
  out_types = jax.ShapeDtypeStruct((NCORES, NP, DEGW), _F32)

  scratch = [
      pltpu.VMEM((NBUF, CHUNK), jnp.int32),
      pltpu.VMEM((CHUNK, DEGW), _F32),
      pltpu.VMEM_SHARED((NP, DEGW), _F32),
      pltpu.SemaphoreType.DMA((NBUF,)),
  ]

  def body(dst_hbm, zd_hbm, od_hbm, deg_out, ring, ones_v, deg_sh, sems):
    cid = lax.axis_index("c")
    sid = lax.axis_index("s")
    tile = cid * NSUB + sid
    rb = sid * ROWS_PER_TILE
    ebase = tile * NCHUNK

    pltpu.sync_copy(zd_hbm.at[pl.ds(rb, ROWS_PER_TILE)],
                    deg_sh.at[pl.ds(rb, ROWS_PER_TILE)])
    pltpu.sync_copy(od_hbm, ones_v)

    plsc.subcore_barrier()

    for b in range(NBUF):
      pltpu.async_copy(dst_hbm.at[ebase + b], ring.at[b], sems.at[b])

    @pl.loop(0, NCHUNK, step=NBUF)
    def _(g):
      for b in range(NBUF):
        ci = g + b
        pltpu.make_async_copy(
            dst_hbm.at[ebase + ci], ring.at[b], sems.at[b]).wait()
        pltpu.sync_copy(ones_v, deg_sh.at[ring.at[b]], add=True)

        nci = ci + NBUF

        @pl.when(nci < NCHUNK)
        def _():
          pltpu.async_copy(dst_hbm.at[ebase + nci], ring.at[b], sems.at[b])

    plsc.subcore_barrier()

    pltpu.sync_copy(deg_sh.at[pl.ds(rb, ROWS_PER_TILE)],
                    deg_out.at[cid, pl.ds(rb, ROWS_PER_TILE)])

  k = pl.kernel(body, out_type=out_types, mesh=_sc_mesh(),
                scratch_types=scratch)
  return k(dst2d, zeros_deg, ones_deg)


def _mm_kernel(x_ref, w_ref, o_ref):
  o_ref[...] = jnp.dot(x_ref[...], w_ref[...],
                       preferred_element_type=_F32,
                       precision=lax.Precision.HIGHEST)


def _tc_matmul(x, w):
  br = 632
  return pl.pallas_call(
      _mm_kernel,
      grid=(NP // br,),
      in_specs=[
          pl.BlockSpec((br, D), lambda i: (i, 0)),
          pl.BlockSpec((D, D), lambda i: (0, 0)),
      ],
      out_specs=pl.BlockSpec((br, D), lambda i: (i, 0)),
      out_shape=jax.ShapeDtypeStruct((NP, D), _F32),
  )(x, w)


def _mid_kernel(a_ref, d_ref, b_ref, w_ref, o_ref):
  deg = d_ref[0, :, 0:1] + d_ref[1, :, 0:1]
  norm = 1.0 / jnp.maximum(deg, 1.0)
  h = (a_ref[0] + a_ref[1]) * norm + b_ref[...]
  h = jnp.maximum(h, 0.0)
  o_ref[...] = jnp.dot(h, w_ref[...], preferred_element_type=_F32,
                       precision=lax.Precision.HIGHEST)


def _tc_mid(agg, deg, b1, w2):
  br = 632
  return pl.pallas_call(
      _mid_kernel,
      grid=(NP // br,),
      in_specs=[
          pl.BlockSpec((2, br, D), lambda i: (0, i, 0)),
          pl.BlockSpec((2, br, DEGW), lambda i: (0, i, 0)),
          pl.BlockSpec((1, D), lambda i: (0, 0)),
          pl.BlockSpec((D, D), lambda i: (0, 0)),
      ],
      out_specs=pl.BlockSpec((br, D), lambda i: (i, 0)),
      out_shape=jax.ShapeDtypeStruct((NP, D), _F32),
  )(agg, deg, b1, w2)


def _final_kernel(a_ref, d_ref, b_ref, o_ref):
  deg = d_ref[0, :, 0:1] + d_ref[1, :, 0:1]
  norm = 1.0 / jnp.maximum(deg, 1.0)
  o_ref[...] = (a_ref[0] + a_ref[1]) * norm + b_ref[...]


def _tc_final(agg, deg, b2):
  br = 632
  return pl.pallas_call(
      _final_kernel,
      grid=(NP // br,),
      in_specs=[
          pl.BlockSpec((2, br, D), lambda i: (0, i, 0)),
          pl.BlockSpec((2, br, DEGW), lambda i: (0, i, 0)),
          pl.BlockSpec((1, D), lambda i: (0, 0)),
      ],
      out_specs=pl.BlockSpec((br, D), lambda i: (i, 0)),
      out_shape=jax.ShapeDtypeStruct((NP, D), _F32),
  )(agg, deg, b2)


def kernel(edge_index, features, W1, b1, W2, b2):
  src = edge_index[0].astype(jnp.int32)
  dst = edge_index[1].astype(jnp.int32)
  pad = jnp.full((EP - E,), N, jnp.int32)
  src2d = jnp.concatenate([src, pad]).reshape(EP // CHUNK, CHUNK)
  dst2d = jnp.concatenate([dst, pad]).reshape(EP // CHUNK, CHUNK)
  idx2d = jnp.stack([src2d, dst2d], axis=1)

  x = jnp.zeros((NP, D), _F32).at[:N].set(features)
  b1r = b1.reshape(1, D)
  b2r = b2.reshape(1, D)

  zeros_big = jnp.zeros((NP, D), _F32)
  zeros_deg = zeros_big
  ones_deg = jnp.ones((CHUNK, DEGW), _F32)

  deg = _sc_degree(dst2d, zeros_deg, ones_deg)
  hw1 = _tc_matmul(x, W1)
  agg1 = _sc_aggregate(hw1, idx2d, zeros_big)
  hw2 = _tc_mid(agg1, deg, b1r, W2)
  agg2 = _sc_aggregate(hw2, idx2d, zeros_big)
  out = _tc_final(agg2, deg, b2r)
  return out[:N]


# speedup vs baseline: 1.2642x; 1.2642x over previous
import jax
import jax.numpy as jnp
from jax import lax
from jax.experimental import pallas as pl
from jax.experimental.pallas import tpu as pltpu
from jax.experimental.pallas import tpu_sc as plsc

N = 10000
NP = 10112
E = 320000
EP = 327680
D = 128
NCORES = 2
NSUB = 16
CHUNK = 128
NCHUNKS_TOTAL = EP // CHUNK
NCHUNK0 = 118
NCHUNK1 = NCHUNKS_TOTAL // NSUB - NCHUNK0
NCHUNK = EP // (NCORES * NSUB * CHUNK)
ROWS_PER_TILE = NP // NSUB
NBUF = 2
DEGW = 128

_F32 = jnp.float32


def _sc_mesh():
  return plsc.VectorSubcoreMesh(core_axis_name="c", subcore_axis_name="s")


def _sc_aggregate(hw, idx2d, zeros_big):
  out_types = jax.ShapeDtypeStruct((NCORES, NP, D), _F32)

  scratch = [
      pltpu.VMEM((NBUF, 2, CHUNK), jnp.int32),
      pltpu.VMEM((NBUF, CHUNK, D), _F32),
      pltpu.VMEM_SHARED((NP, D), _F32),
      pltpu.SemaphoreType.DMA((NBUF,)),
  ]

  def body(hw_hbm, idx_hbm, zb_hbm, acc_out, idx_v, rows, acc_sh, sems):
    cid = lax.axis_index("c")
    sid = lax.axis_index("s")
    rb = sid * ROWS_PER_TILE

    pltpu.sync_copy(zb_hbm.at[pl.ds(rb, ROWS_PER_TILE)],
                    acc_sh.at[pl.ds(rb, ROWS_PER_TILE)])

    plsc.subcore_barrier()

    def run_ring(ebase, nchunk):
      if nchunk == 0:
        return
      for b in range(NBUF):
        pltpu.sync_copy(idx_hbm.at[ebase + b], idx_v.at[b])
        pltpu.async_copy(hw_hbm.at[idx_v.at[b, 0]], rows.at[b], sems.at[b])

      @pl.loop(0, nchunk, step=NBUF)
      def _(g):
        for b in range(NBUF):
          ci = g + b
          pltpu.make_async_copy(
              hw_hbm.at[idx_v.at[b, 0]], rows.at[b], sems.at[b]).wait()
          pltpu.sync_copy(rows.at[b], acc_sh.at[idx_v.at[b, 1]], add=True)

          nci = ci + NBUF

          @pl.when(nci < nchunk)
          def _():
            pltpu.sync_copy(idx_hbm.at[ebase + nci], idx_v.at[b])
            pltpu.async_copy(hw_hbm.at[idx_v.at[b, 0]], rows.at[b],
                             sems.at[b])

    @pl.when(cid == 0)
    def _():
      run_ring(sid * NCHUNK0, NCHUNK0)

    @pl.when(cid == 1)
    def _():
      run_ring(NSUB * NCHUNK0 + sid * NCHUNK1, NCHUNK1)

    plsc.subcore_barrier()

    pltpu.sync_copy(acc_sh.at[pl.ds(rb, ROWS_PER_TILE)],
                    acc_out.at[cid, pl.ds(rb, ROWS_PER_TILE)])

  k = pl.kernel(body, out_type=out_types, mesh=_sc_mesh(),
                scratch_types=scratch)
  return k(hw, idx2d, zeros_big)


def _sc_degree(dst2d, zeros_deg, ones_deg):
  out_types = jax.ShapeDtypeStruct((NCORES, NP, DEGW), _F32)

  scratch = [
      pltpu.VMEM((NBUF, CHUNK), jnp.int32),
      pltpu.VMEM((CHUNK, DEGW), _F32),
      pltpu.VMEM_SHARED((NP, DEGW), _F32),
      pltpu.SemaphoreType.DMA((NBUF,)),
  ]

  def body(dst_hbm, zd_hbm, od_hbm, deg_out, ring, ones_v, deg_sh, sems):
    cid = lax.axis_index("c")
    sid = lax.axis_index("s")
    tile = cid * NSUB + sid
    rb = sid * ROWS_PER_TILE
    ebase = tile * NCHUNK

    pltpu.sync_copy(zd_hbm.at[pl.ds(rb, ROWS_PER_TILE)],
                    deg_sh.at[pl.ds(rb, ROWS_PER_TILE)])
    pltpu.sync_copy(od_hbm, ones_v)

    plsc.subcore_barrier()

    for b in range(NBUF):
      pltpu.async_copy(dst_hbm.at[ebase + b], ring.at[b], sems.at[b])

    @pl.loop(0, NCHUNK, step=NBUF)
    def _(g):
      for b in range(NBUF):
        ci = g + b
        pltpu.make_async_copy(
            dst_hbm.at[ebase + ci], ring.at[b], sems.at[b]).wait()
        pltpu.sync_copy(ones_v, deg_sh.at[ring.at[b]], add=True)

        nci = ci + NBUF

        @pl.when(nci < NCHUNK)
        def _():
          pltpu.async_copy(dst_hbm.at[ebase + nci], ring.at[b], sems.at[b])

    plsc.subcore_barrier()

    pltpu.sync_copy(deg_sh.at[pl.ds(rb, ROWS_PER_TILE)],
                    deg_out.at[cid, pl.ds(rb, ROWS_PER_TILE)])

  k = pl.kernel(body, out_type=out_types, mesh=_sc_mesh(),
                scratch_types=scratch)
  return k(dst2d, zeros_deg, ones_deg)


def _mm_kernel(x_ref, w_ref, o_ref):
  o_ref[...] = jnp.dot(x_ref[...], w_ref[...],
                       preferred_element_type=_F32,
                       precision=lax.Precision.HIGHEST)


def _tc_matmul(x, w):
  br = 632
  return pl.pallas_call(
      _mm_kernel,
      grid=(NP // br,),
      in_specs=[
          pl.BlockSpec((br, D), lambda i: (i, 0)),
          pl.BlockSpec((D, D), lambda i: (0, 0)),
      ],
      out_specs=pl.BlockSpec((br, D), lambda i: (i, 0)),
      out_shape=jax.ShapeDtypeStruct((NP, D), _F32),
  )(x, w)


def _mid_kernel(a_ref, d_ref, b_ref, w_ref, o_ref):
  deg = d_ref[0, :, 0:1] + d_ref[1, :, 0:1]
  norm = 1.0 / jnp.maximum(deg, 1.0)
  h = (a_ref[0] + a_ref[1]) * norm + b_ref[...]
  h = jnp.maximum(h, 0.0)
  o_ref[...] = jnp.dot(h, w_ref[...], preferred_element_type=_F32,
                       precision=lax.Precision.HIGHEST)


def _tc_mid(agg, deg, b1, w2):
  br = 632
  return pl.pallas_call(
      _mid_kernel,
      grid=(NP // br,),
      in_specs=[
          pl.BlockSpec((2, br, D), lambda i: (0, i, 0)),
          pl.BlockSpec((2, br, DEGW), lambda i: (0, i, 0)),
          pl.BlockSpec((1, D), lambda i: (0, 0)),
          pl.BlockSpec((D, D), lambda i: (0, 0)),
      ],
      out_specs=pl.BlockSpec((br, D), lambda i: (i, 0)),
      out_shape=jax.ShapeDtypeStruct((NP, D), _F32),
  )(agg, deg, b1, w2)


def _final_kernel(a_ref, d_ref, b_ref, o_ref):
  deg = d_ref[0, :, 0:1] + d_ref[1, :, 0:1]
  norm = 1.0 / jnp.maximum(deg, 1.0)
  o_ref[...] = (a_ref[0] + a_ref[1]) * norm + b_ref[...]


def _tc_final(agg, deg, b2):
  br = 632
  return pl.pallas_call(
      _final_kernel,
      grid=(NP // br,),
      in_specs=[
          pl.BlockSpec((2, br, D), lambda i: (0, i, 0)),
          pl.BlockSpec((2, br, DEGW), lambda i: (0, i, 0)),
          pl.BlockSpec((1, D), lambda i: (0, 0)),
      ],
      out_specs=pl.BlockSpec((br, D), lambda i: (i, 0)),
      out_shape=jax.ShapeDtypeStruct((NP, D), _F32),
  )(agg, deg, b2)


def kernel(edge_index, features, W1, b1, W2, b2):
  src = edge_index[0].astype(jnp.int32)
  dst = edge_index[1].astype(jnp.int32)
  pad = jnp.full((EP - E,), N, jnp.int32)
  src2d = jnp.concatenate([src, pad]).reshape(EP // CHUNK, CHUNK)
  dst2d = jnp.concatenate([dst, pad]).reshape(EP // CHUNK, CHUNK)
  idx2d = jnp.stack([src2d, dst2d], axis=1)

  x = jnp.zeros((NP, D), _F32).at[:N].set(features)
  b1r = b1.reshape(1, D)
  b2r = b2.reshape(1, D)

  zeros_big = jnp.zeros((NP, D), _F32)
  zeros_deg = zeros_big
  ones_deg = jnp.ones((CHUNK, DEGW), _F32)

  deg = _sc_degree(dst2d, zeros_deg, ones_deg)
  hw1 = _tc_matmul(x, W1)
  agg1 = _sc_aggregate(hw1, idx2d, zeros_big)
  hw2 = _tc_mid(agg1, deg, b1r, W2)
  agg2 = _sc_aggregate(hw2, idx2d, zeros_big)
  out = _tc_final(agg2, deg, b2r)
  return out[:N]


# speedup vs baseline: 1.2651x; 1.0007x over previous
import jax
import jax.numpy as jnp
from jax import lax
from jax.experimental import pallas as pl
from jax.experimental.pallas import tpu as pltpu
from jax.experimental.pallas import tpu_sc as plsc

N = 10000
NP = 10112
E = 320000
EP = 327680
D = 128
NCORES = 2
NSUB = 16
CHUNK = 128
NCHUNKS_TOTAL = EP // CHUNK
NCHUNK0 = 118
NCHUNK1 = NCHUNKS_TOTAL // NSUB - NCHUNK0
NCHUNK = EP // (NCORES * NSUB * CHUNK)
ROWS_PER_TILE = NP // NSUB
NBUF = 2
DEGW = 128

_F32 = jnp.float32


def _sc_mesh():
  return plsc.VectorSubcoreMesh(core_axis_name="c", subcore_axis_name="s")


def _sc_aggregate(hw, idx2d, zeros_big):
  out_types = jax.ShapeDtypeStruct((NCORES, NP, D), _F32)

  scratch = [
      pltpu.VMEM((NBUF, 2, CHUNK), jnp.int32),
      pltpu.VMEM((NBUF, CHUNK, D), _F32),
      pltpu.VMEM_SHARED((NP, D), _F32),
      pltpu.SemaphoreType.DMA((NBUF,)),
  ]

  def body(hw_hbm, idx_hbm, zb_hbm, acc_out, idx_v, rows, acc_sh, sems):
    cid = lax.axis_index("c")
    sid = lax.axis_index("s")
    rb = sid * ROWS_PER_TILE

    pltpu.sync_copy(zb_hbm.at[pl.ds(rb, ROWS_PER_TILE)],
                    acc_sh.at[pl.ds(rb, ROWS_PER_TILE)])

    plsc.subcore_barrier()

    def run_ring(ebase, nchunk):
      if nchunk == 0:
        return
      for b in range(NBUF):
        pltpu.sync_copy(idx_hbm.at[ebase + b], idx_v.at[b])
        pltpu.async_copy(hw_hbm.at[idx_v.at[b, 0]], rows.at[b], sems.at[b])

      @pl.loop(0, nchunk, step=NBUF)
      def _(g):
        for b in range(NBUF):
          ci = g + b
          pltpu.make_async_copy(
              hw_hbm.at[idx_v.at[b, 0]], rows.at[b], sems.at[b]).wait()
          pltpu.sync_copy(rows.at[b], acc_sh.at[idx_v.at[b, 1]], add=True)

          nci = ci + NBUF

          @pl.when(nci < nchunk)
          def _():
            pltpu.sync_copy(idx_hbm.at[ebase + nci], idx_v.at[b])
            pltpu.async_copy(hw_hbm.at[idx_v.at[b, 0]], rows.at[b],
                             sems.at[b])

    @pl.when(cid == 0)
    def _():
      run_ring(sid * NCHUNK0, NCHUNK0)

    @pl.when(cid == 1)
    def _():
      run_ring(NSUB * NCHUNK0 + sid * NCHUNK1, NCHUNK1)

    plsc.subcore_barrier()

    pltpu.sync_copy(acc_sh.at[pl.ds(rb, ROWS_PER_TILE)],
                    acc_out.at[cid, pl.ds(rb, ROWS_PER_TILE)])

  k = pl.kernel(body, out_type=out_types, mesh=_sc_mesh(),
                scratch_types=scratch)
  return k(hw, idx2d, zeros_big)


def _sc_degree(dst2d, zeros_deg):
  out_types = jax.ShapeDtypeStruct((NCORES, NP, DEGW), _F32)

  scratch = [
      pltpu.VMEM((NBUF, CHUNK), jnp.int32),
      pltpu.VMEM((CHUNK, DEGW), _F32),
      pltpu.VMEM_SHARED((NP, DEGW), _F32),
      pltpu.SemaphoreType.DMA((NBUF,)),
  ]

  def body(dst_hbm, zd_hbm, deg_out, ring, ones_v, deg_sh, sems):
    cid = lax.axis_index("c")
    sid = lax.axis_index("s")
    tile = cid * NSUB + sid
    rb = sid * ROWS_PER_TILE
    ebase = tile * NCHUNK

    pltpu.sync_copy(zd_hbm.at[pl.ds(rb, ROWS_PER_TILE)],
                    deg_sh.at[pl.ds(rb, ROWS_PER_TILE)])

    @pl.loop(0, CHUNK)
    def _(r):
      @pl.loop(0, DEGW, step=16)
      def _(c0):
        ones_v[r, pl.ds(c0, 16)] = jnp.ones((16,), _F32)

    plsc.subcore_barrier()

    for b in range(NBUF):
      pltpu.async_copy(dst_hbm.at[ebase + b], ring.at[b], sems.at[b])

    @pl.loop(0, NCHUNK, step=NBUF)
    def _(g):
      for b in range(NBUF):
        ci = g + b
        pltpu.make_async_copy(
            dst_hbm.at[ebase + ci], ring.at[b], sems.at[b]).wait()
        pltpu.sync_copy(ones_v, deg_sh.at[ring.at[b]], add=True)

        nci = ci + NBUF

        @pl.when(nci < NCHUNK)
        def _():
          pltpu.async_copy(dst_hbm.at[ebase + nci], ring.at[b], sems.at[b])

    plsc.subcore_barrier()

    pltpu.sync_copy(deg_sh.at[pl.ds(rb, ROWS_PER_TILE)],
                    deg_out.at[cid, pl.ds(rb, ROWS_PER_TILE)])

  k = pl.kernel(body, out_type=out_types, mesh=_sc_mesh(),
                scratch_types=scratch)
  return k(dst2d, zeros_deg)


def _mm_kernel(x_ref, w_ref, o_ref):
  o_ref[...] = jnp.dot(x_ref[...], w_ref[...],
                       preferred_element_type=_F32,
                       precision=lax.Precision.HIGHEST)


def _tc_matmul(x, w):
  br = 632
  return pl.pallas_call(
      _mm_kernel,
      grid=(NP // br,),
      in_specs=[
          pl.BlockSpec((br, D), lambda i: (i, 0)),
          pl.BlockSpec((D, D), lambda i: (0, 0)),
      ],
      out_specs=pl.BlockSpec((br, D), lambda i: (i, 0)),
      out_shape=jax.ShapeDtypeStruct((NP, D), _F32),
  )(x, w)


def _mid_kernel(a_ref, d_ref, b_ref, w_ref, o_ref):
  deg = d_ref[0, :, 0:1] + d_ref[1, :, 0:1]
  norm = 1.0 / jnp.maximum(deg, 1.0)
  h = (a_ref[0] + a_ref[1]) * norm + b_ref[...]
  h = jnp.maximum(h, 0.0)
  o_ref[...] = jnp.dot(h, w_ref[...], preferred_element_type=_F32,
                       precision=lax.Precision.HIGHEST)


def _tc_mid(agg, deg, b1, w2):
  br = 632
  return pl.pallas_call(
      _mid_kernel,
      grid=(NP // br,),
      in_specs=[
          pl.BlockSpec((2, br, D), lambda i: (0, i, 0)),
          pl.BlockSpec((2, br, DEGW), lambda i: (0, i, 0)),
          pl.BlockSpec((1, D), lambda i: (0, 0)),
          pl.BlockSpec((D, D), lambda i: (0, 0)),
      ],
      out_specs=pl.BlockSpec((br, D), lambda i: (i, 0)),
      out_shape=jax.ShapeDtypeStruct((NP, D), _F32),
  )(agg, deg, b1, w2)


def _final_kernel(a_ref, d_ref, b_ref, o_ref):
  deg = d_ref[0, :, 0:1] + d_ref[1, :, 0:1]
  norm = 1.0 / jnp.maximum(deg, 1.0)
  o_ref[...] = (a_ref[0] + a_ref[1]) * norm + b_ref[...]


def _tc_final(agg, deg, b2):
  br = 632
  return pl.pallas_call(
      _final_kernel,
      grid=(NP // br,),
      in_specs=[
          pl.BlockSpec((2, br, D), lambda i: (0, i, 0)),
          pl.BlockSpec((2, br, DEGW), lambda i: (0, i, 0)),
          pl.BlockSpec((1, D), lambda i: (0, 0)),
      ],
      out_specs=pl.BlockSpec((br, D), lambda i: (i, 0)),
      out_shape=jax.ShapeDtypeStruct((NP, D), _F32),
  )(agg, deg, b2)


def kernel(edge_index, features, W1, b1, W2, b2):
  src = edge_index[0].astype(jnp.int32)
  dst = edge_index[1].astype(jnp.int32)
  pad = jnp.full((EP - E,), N, jnp.int32)
  src2d = jnp.concatenate([src, pad]).reshape(EP // CHUNK, CHUNK)
  dst2d = jnp.concatenate([dst, pad]).reshape(EP // CHUNK, CHUNK)
  idx2d = jnp.stack([src2d, dst2d], axis=1)

  x = jnp.zeros((NP, D), _F32).at[:N].set(features)
  b1r = b1.reshape(1, D)
  b2r = b2.reshape(1, D)

  zeros_big = jnp.zeros((NP, D), _F32)
  zeros_deg = zeros_big

  hw1 = _tc_matmul(x, W1)
  deg = _sc_degree(dst2d, zeros_deg)
  agg1 = _sc_aggregate(hw1, idx2d, zeros_big)
  hw2 = _tc_mid(agg1, deg, b1r, W2)
  agg2 = _sc_aggregate(hw2, idx2d, zeros_big)
  out = _tc_final(agg2, deg, b2r)
  return out[:N]
